# baseline (speedup 1.0000x reference)
import functools

import jax
import jax.numpy as jnp
import numpy as np
from jax import lax
from jax.experimental import pallas as pl
from jax.experimental.pallas import tpu as pltpu
from jax.experimental.pallas import tpu_sc as plsc

NC = 2
NS = 16
LANES = 16


def _sc_fm(idx3, emb2d, lin1d, x1d, combo, B, F, D, R):
    NW = NC * NS
    bpw = B // NW
    CH = bpw // R
    IPC = R * F
    G = D // LANES
    V = lin1d.shape[0]
    NFULL = IPC // LANES
    NBLK = bpw // LANES
    CSZ = combo.shape[0]

    mesh = plsc.VectorSubcoreMesh(core_axis_name="c", subcore_axis_name="s")

    @functools.partial(
        pl.kernel,
        out_type=jax.ShapeDtypeStruct((B,), jnp.float32),
        mesh=mesh,
        compiler_params=pltpu.CompilerParams(use_tc_tiling_on_sc=False,
                                             needs_layout_passes=False),
        scratch_types=[
            pltpu.VMEM((CH, IPC), jnp.int32),
            pltpu.VMEM((2, IPC, D), jnp.bfloat16),
            pltpu.VMEM((V,), jnp.float32),
            pltpu.VMEM((bpw * D,), jnp.float32),
            pltpu.VMEM((bpw * LANES,), jnp.float32),
            pltpu.VMEM((bpw,), jnp.float32),
            pltpu.VMEM((CSZ,), jnp.float32),
            pltpu.VMEM((bpw,), jnp.float32),
            pltpu.SemaphoreType.DMA,
            pltpu.SemaphoreType.DMA,
            pltpu.SemaphoreType.DMA,
        ],
    )
    def body(idx_hbm, emb_hbm, lin_hbm, x_hbm, combo_hbm, out_hbm,
             idx_v, emb_b, lin_v, s_loc, q_loc, x_v, combo_v, out_v,
             sem0, sem1, seml):
        wid = lax.axis_index("s") * NC + lax.axis_index("c")
        pltpu.async_copy(lin_hbm, lin_v, seml)
        pltpu.sync_copy(idx_hbm.at[wid], idx_v)
        pltpu.sync_copy(x_hbm.at[pl.ds(wid * bpw, bpw)], x_v)
        pltpu.sync_copy(combo_hbm, combo_v)
        sems = (sem0, sem1)

        def issue(j, slot):
            pltpu.async_copy(emb_hbm.at[idx_v.at[j]], emb_b.at[slot],
                             sems[slot])

        def drain(j, slot):
            pltpu.make_async_copy(emb_hbm.at[idx_v.at[j]], emb_b.at[slot],
                                  sems[slot]).wait()

        def compute(j, slot):
            for r in range(R):
                q = None
                accs = [None] * G
                for f in range(F):
                    for h in range(D // 32):
                        vab = emb_b[slot, f * R + r, pl.ds(h * 32, 32)]
                        a, b = plsc.unpack(
                            vab, format=plsc.PackFormat.INTERLEAVED)
                        ga, gb = 2 * h, 2 * h + 1
                        accs[ga] = a if accs[ga] is None else accs[ga] + a
                        accs[gb] = b if accs[gb] is None else accs[gb] + b
                        qq = a * a + b * b
                        q = qq if q is None else q + qq
                for g in range(G):
                    s_loc[pl.ds((j * R + r) * D + g * LANES, LANES)] = accs[g]
                q_loc[pl.ds((j * R + r) * LANES, LANES)] = q

        issue(0, 0)
        issue(1, 1)

        @pl.loop(0, CH, step=2)
        def _(j0):
            drain(j0, 0)
            compute(j0, 0)

            @pl.when(j0 + 2 < CH)
            def _():
                issue(j0 + 2, 0)

            drain(j0 + 1, 1)
            compute(j0 + 1, 1)

            @pl.when(j0 + 3 < CH)
            def _():
                issue(j0 + 3, 1)

        pltpu.make_async_copy(lin_hbm, lin_v, seml).wait()
        lanes = lax.broadcasted_iota(jnp.int32, (LANES,), 0)
        tail_keep = lanes >= (NFULL * LANES - (IPC - LANES))
        zeros = jnp.zeros((LANES,), jnp.float32)

        @pl.loop(0, CH)
        def _(j):
            lsum = None
            for m in range(NFULL):
                iv = idx_v[j, pl.ds(m * LANES, LANES)]
                vals = plsc.load_gather(lin_v, [iv])
                lsum = vals if lsum is None else lsum + vals
            if NFULL * LANES < IPC:
                iv = idx_v[j, pl.ds(IPC - LANES, LANES)]
                vals = plsc.load_gather(lin_v, [iv])
                lsum = lsum + jnp.where(tail_keep, vals, zeros)
            for r in range(R):
                contrib = jnp.where(lanes % R == r, lsum, zeros)
                base = (j * R + r) * LANES
                q_loc[pl.ds(base, LANES)] = (
                    q_loc[pl.ds(base, LANES)] - 2.0 * contrib)

        scal = combo_v[pl.ds(2 * D, LANES)]
        w_lin = scal[0]
        b_lin = scal[1]
        bias = scal[2]
        wv = [combo_v[pl.ds(g * LANES, LANES)] for g in range(G)]
        cv = [combo_v[pl.ds(D + g * LANES, LANES)] for g in range(G)]
        for blk in range(NBLK):
            xv = x_v[pl.ds(blk * LANES, LANES)]
            acc = zeros
            ecsq = zeros
            for d in range(D):
                sd = plsc.load_gather(
                    s_loc, [lanes * D + (blk * LANES * D + d)])
                ec = xv * wv[d // LANES][d % LANES] + cv[d // LANES][d % LANES]
                t = sd + ec
                acc = acc + t * t
                ecsq = ecsq + ec * ec
            qsum = zeros
            for k in range(LANES):
                qk = plsc.load_gather(
                    q_loc, [lanes * LANES + (blk * LANES * LANES + k)])
                qsum = qsum + qk
            out_v[pl.ds(blk * LANES, LANES)] = (
                0.5 * (acc - qsum - ecsq) + bias + xv * w_lin + b_lin)

        pltpu.sync_copy(out_v, out_hbm.at[pl.ds(wid * bpw, bpw)])

    return body(idx3, emb2d, lin1d, x1d, combo)


def kernel(categorical_features, continuous_features, linear_table, bias,
           embedding_table, W_lin, b_lin, W_emb, b_emb):
    B, F = categorical_features.shape
    V, D = embedding_table.shape
    NW = NC * NS
    R = 4
    CH = (B // NW) // R

    idx3 = (categorical_features.astype(jnp.int32)
            .reshape(NW, CH, R, F).transpose(0, 1, 3, 2)
            .reshape(NW, CH, F * R))
    embf = embedding_table.astype(jnp.bfloat16)
    emb2d = lax.optimization_barrier(embf.reshape(-1)).reshape(V, D)
    lin1d = linear_table.reshape(-1)
    x1d = continuous_features.reshape(-1)
    perm = np.empty((D,), np.int32)
    for h in range(D // 32):
        for k in range(16):
            perm[32 * h + k] = 32 * h + 2 * k
            perm[32 * h + 16 + k] = 32 * h + 2 * k + 1
    permj = jnp.asarray(perm)
    combo = jnp.concatenate([
        W_emb.reshape(-1)[permj], b_emb.reshape(-1)[permj],
        W_lin.reshape(-1), b_lin.reshape(-1), bias.reshape(-1),
        jnp.zeros((2 * D + 16 - (2 * D + 3),), jnp.float32),
    ])
    return _sc_fm(idx3, emb2d, lin1d, x1d, combo, B, F, D, R)


# speedup vs baseline: 1.0004x; 1.0004x over previous
import functools

import jax
import jax.numpy as jnp
import numpy as np
from jax import lax
from jax.experimental import pallas as pl
from jax.experimental.pallas import tpu as pltpu
from jax.experimental.pallas import tpu_sc as plsc

NC = 2
NS = 16
LANES = 16


def _sc_fm(idx3, emb2d, lin1d, x1d, combo, B, F, D, R):
    NW = NC * NS
    bpw = B // NW
    CH = bpw // R
    IPC = R * F
    G = D // LANES
    V = lin1d.shape[0]
    NFULL = IPC // LANES
    NBLK = bpw // LANES
    CSZ = combo.shape[0]

    mesh = plsc.VectorSubcoreMesh(core_axis_name="c", subcore_axis_name="s")

    @functools.partial(
        pl.kernel,
        out_type=jax.ShapeDtypeStruct((B,), jnp.float32),
        mesh=mesh,
        compiler_params=pltpu.CompilerParams(use_tc_tiling_on_sc=False,
                                             needs_layout_passes=False),
        scratch_types=[
            pltpu.VMEM((CH, IPC), jnp.int32),
            pltpu.VMEM((2, IPC, D), jnp.bfloat16),
            pltpu.VMEM((V,), jnp.float32),
            pltpu.VMEM((bpw * D,), jnp.float32),
            pltpu.VMEM((bpw * LANES,), jnp.float32),
            pltpu.VMEM((bpw,), jnp.float32),
            pltpu.VMEM((CSZ,), jnp.float32),
            pltpu.VMEM((bpw,), jnp.float32),
            pltpu.SemaphoreType.DMA,
            pltpu.SemaphoreType.DMA,
            pltpu.SemaphoreType.DMA,
        ],
    )
    def body(idx_hbm, emb_hbm, lin_hbm, x_hbm, combo_hbm, out_hbm,
             idx_v, emb_b, lin_v, s_loc, q_loc, x_v, combo_v, out_v,
             sem0, sem1, seml):
        wid = lax.axis_index("s") * NC + lax.axis_index("c")
        pltpu.async_copy(lin_hbm, lin_v, seml)
        pltpu.sync_copy(idx_hbm.at[wid], idx_v)
        pltpu.sync_copy(x_hbm.at[pl.ds(wid * bpw, bpw)], x_v)
        pltpu.sync_copy(combo_hbm, combo_v)
        sems = (sem0, sem1)

        def issue(j, slot):
            pltpu.async_copy(emb_hbm.at[idx_v.at[j]], emb_b.at[slot],
                             sems[slot])

        def drain(j, slot):
            pltpu.make_async_copy(emb_hbm.at[idx_v.at[j]], emb_b.at[slot],
                                  sems[slot]).wait()

        def compute(j, slot):
            for r in range(R):
                q = None
                accs = [None] * G
                for f in range(F):
                    for h in range(D // 32):
                        vab = emb_b[slot, f * R + r, pl.ds(h * 32, 32)]
                        a, b = plsc.unpack(
                            vab, format=plsc.PackFormat.INTERLEAVED)
                        ga, gb = 2 * h, 2 * h + 1
                        accs[ga] = a if accs[ga] is None else accs[ga] + a
                        accs[gb] = b if accs[gb] is None else accs[gb] + b
                        qq = a * a + b * b
                        q = qq if q is None else q + qq
                for g in range(G):
                    s_loc[pl.ds((j * R + r) * D + g * LANES, LANES)] = accs[g]
                q_loc[pl.ds((j * R + r) * LANES, LANES)] = q

        issue(0, 0)
        issue(1, 1)

        @pl.loop(0, CH, step=2)
        def _(j0):
            drain(j0, 0)
            compute(j0, 0)

            @pl.when(j0 + 2 < CH)
            def _():
                issue(j0 + 2, 0)

            drain(j0 + 1, 1)
            compute(j0 + 1, 1)

            @pl.when(j0 + 3 < CH)
            def _():
                issue(j0 + 3, 1)

        pltpu.make_async_copy(lin_hbm, lin_v, seml).wait()
        lanes = lax.broadcasted_iota(jnp.int32, (LANES,), 0)
        tail_keep = lanes >= (NFULL * LANES - (IPC - LANES))
        zeros = jnp.zeros((LANES,), jnp.float32)

        @pl.loop(0, CH)
        def _(j):
            lsum = None
            for m in range(NFULL):
                iv = idx_v[j, pl.ds(m * LANES, LANES)]
                vals = plsc.load_gather(lin_v, [iv])
                lsum = vals if lsum is None else lsum + vals
            if NFULL * LANES < IPC:
                iv = idx_v[j, pl.ds(IPC - LANES, LANES)]
                vals = plsc.load_gather(lin_v, [iv])
                lsum = lsum + jnp.where(tail_keep, vals, zeros)
            for r in range(R):
                contrib = jnp.where(lanes % R == r, lsum, zeros)
                base = (j * R + r) * LANES
                q_loc[pl.ds(base, LANES)] = (
                    q_loc[pl.ds(base, LANES)] - 2.0 * contrib)

        scal = combo_v[pl.ds(2 * D, LANES)]
        w_lin = scal[0]
        b_lin = scal[1]
        bias = scal[2]
        wv = [combo_v[pl.ds(g * LANES, LANES)] for g in range(G)]
        cv = [combo_v[pl.ds(D + g * LANES, LANES)] for g in range(G)]
        for blk in range(NBLK):
            xv = x_v[pl.ds(blk * LANES, LANES)]
            acc = zeros
            ecsq = zeros
            for d in range(D):
                sd = plsc.load_gather(
                    s_loc, [lanes * D + (blk * LANES * D + d)])
                ec = xv * wv[d // LANES][d % LANES] + cv[d // LANES][d % LANES]
                t = sd + ec
                acc = acc + t * t
                ecsq = ecsq + ec * ec
            qsum = zeros
            for k in range(LANES):
                qk = plsc.load_gather(
                    q_loc, [lanes * LANES + (blk * LANES * LANES + k)])
                qsum = qsum + qk
            out_v[pl.ds(blk * LANES, LANES)] = (
                0.5 * (acc - qsum - ecsq) + bias + xv * w_lin + b_lin)

        pltpu.sync_copy(out_v, out_hbm.at[pl.ds(wid * bpw, bpw)])

    return body(idx3, emb2d, lin1d, x1d, combo)


def kernel(categorical_features, continuous_features, linear_table, bias,
           embedding_table, W_lin, b_lin, W_emb, b_emb):
    B, F = categorical_features.shape
    V, D = embedding_table.shape
    NW = NC * NS
    R = 4
    CH = (B // NW) // R

    idx3 = (categorical_features.astype(jnp.int32)
            .reshape(NW, CH, R, F).transpose(0, 1, 3, 2)
            .reshape(NW, CH, F * R))
    emb_lin = lax.optimization_barrier(embedding_table.reshape(-1))
    emb2d = emb_lin.astype(jnp.bfloat16).reshape(V, D)
    lin1d = linear_table.reshape(-1)
    x1d = continuous_features.reshape(-1)
    perm = np.empty((D,), np.int32)
    for h in range(D // 32):
        for k in range(16):
            perm[32 * h + k] = 32 * h + 2 * k
            perm[32 * h + 16 + k] = 32 * h + 2 * k + 1
    permj = jnp.asarray(perm)
    combo = jnp.concatenate([
        W_emb.reshape(-1)[permj], b_emb.reshape(-1)[permj],
        W_lin.reshape(-1), b_lin.reshape(-1), bias.reshape(-1),
        jnp.zeros((2 * D + 16 - (2 * D + 3),), jnp.float32),
    ])
    return _sc_fm(idx3, emb2d, lin1d, x1d, combo, B, F, D, R)


# speedup vs baseline: 1.1842x; 1.1838x over previous
import functools

import jax
import jax.numpy as jnp
from jax import lax
from jax.experimental import pallas as pl
from jax.experimental.pallas import tpu as pltpu
from jax.experimental.pallas import tpu_sc as plsc

NC = 2
NS = 16
LANES = 16


def _sc_fm(idx3, emb2d, lin1d, x1d, combo, B, F, D, R):
    NW = NC * NS
    bpw = B // NW
    CH = bpw // R
    IPC = R * F
    G = D // LANES
    V = lin1d.shape[0]
    NFULL = IPC // LANES
    NBLK = bpw // LANES
    CSZ = combo.shape[0]

    mesh = plsc.VectorSubcoreMesh(core_axis_name="c", subcore_axis_name="s")

    @functools.partial(
        pl.kernel,
        out_type=jax.ShapeDtypeStruct((B,), jnp.float32),
        mesh=mesh,
        compiler_params=pltpu.CompilerParams(use_tc_tiling_on_sc=False,
                                             needs_layout_passes=False),
        scratch_types=[
            pltpu.VMEM((CH, IPC), jnp.int32),
            pltpu.VMEM((2, IPC, D), jnp.float32),
            pltpu.VMEM((V,), jnp.float32),
            pltpu.VMEM((bpw * D,), jnp.float32),
            pltpu.VMEM((bpw * LANES,), jnp.float32),
            pltpu.VMEM((bpw,), jnp.float32),
            pltpu.VMEM((CSZ,), jnp.float32),
            pltpu.VMEM((bpw,), jnp.float32),
            pltpu.SemaphoreType.DMA,
            pltpu.SemaphoreType.DMA,
            pltpu.SemaphoreType.DMA,
        ],
    )
    def body(idx_hbm, emb_hbm, lin_hbm, x_hbm, combo_hbm, out_hbm,
             idx_v, emb_b, lin_v, s_loc, q_loc, x_v, combo_v, out_v,
             sem0, sem1, seml):
        wid = lax.axis_index("s") * NC + lax.axis_index("c")
        pltpu.async_copy(lin_hbm, lin_v, seml)
        pltpu.sync_copy(idx_hbm.at[wid], idx_v)
        pltpu.sync_copy(x_hbm.at[pl.ds(wid * bpw, bpw)], x_v)
        pltpu.sync_copy(combo_hbm, combo_v)
        sems = (sem0, sem1)

        def issue(j, slot):
            pltpu.async_copy(emb_hbm.at[idx_v.at[j]], emb_b.at[slot],
                             sems[slot])

        def drain(j, slot):
            pltpu.make_async_copy(emb_hbm.at[idx_v.at[j]], emb_b.at[slot],
                                  sems[slot]).wait()

        def compute(j, slot):
            for r in range(R):
                q = None
                for g in range(G):
                    acc = None
                    for f in range(F):
                        v = emb_b[slot, f * R + r, pl.ds(g * LANES, LANES)]
                        acc = v if acc is None else acc + v
                        q = v * v if q is None else q + v * v
                    s_loc[pl.ds((j * R + r) * D + g * LANES, LANES)] = acc
                q_loc[pl.ds((j * R + r) * LANES, LANES)] = q

        issue(0, 0)
        issue(1, 1)

        @pl.loop(0, CH, step=2)
        def _(j0):
            drain(j0, 0)
            compute(j0, 0)

            @pl.when(j0 + 2 < CH)
            def _():
                issue(j0 + 2, 0)

            drain(j0 + 1, 1)
            compute(j0 + 1, 1)

            @pl.when(j0 + 3 < CH)
            def _():
                issue(j0 + 3, 1)

        pltpu.make_async_copy(lin_hbm, lin_v, seml).wait()
        lanes = lax.broadcasted_iota(jnp.int32, (LANES,), 0)
        tail_keep = lanes >= (NFULL * LANES - (IPC - LANES))
        zeros = jnp.zeros((LANES,), jnp.float32)

        @pl.loop(0, CH)
        def _(j):
            lsum = None
            for m in range(NFULL):
                iv = idx_v[j, pl.ds(m * LANES, LANES)]
                vals = plsc.load_gather(lin_v, [iv])
                lsum = vals if lsum is None else lsum + vals
            if NFULL * LANES < IPC:
                iv = idx_v[j, pl.ds(IPC - LANES, LANES)]
                vals = plsc.load_gather(lin_v, [iv])
                lsum = lsum + jnp.where(tail_keep, vals, zeros)
            for r in range(R):
                contrib = jnp.where(lanes % R == r, lsum, zeros)
                base = (j * R + r) * LANES
                q_loc[pl.ds(base, LANES)] = (
                    q_loc[pl.ds(base, LANES)] - 2.0 * contrib)

        scal = combo_v[pl.ds(2 * D, LANES)]
        w_lin = scal[0]
        b_lin = scal[1]
        bias = scal[2]
        wv = [combo_v[pl.ds(g * LANES, LANES)] for g in range(G)]
        cv = [combo_v[pl.ds(D + g * LANES, LANES)] for g in range(G)]
        for blk in range(NBLK):
            xv = x_v[pl.ds(blk * LANES, LANES)]
            acc = zeros
            ecsq = zeros
            for d in range(D):
                sd = plsc.load_gather(
                    s_loc, [lanes * D + (blk * LANES * D + d)])
                ec = xv * wv[d // LANES][d % LANES] + cv[d // LANES][d % LANES]
                t = sd + ec
                acc = acc + t * t
                ecsq = ecsq + ec * ec
            qsum = zeros
            for k in range(LANES):
                qk = plsc.load_gather(
                    q_loc, [lanes * LANES + (blk * LANES * LANES + k)])
                qsum = qsum + qk
            out_v[pl.ds(blk * LANES, LANES)] = (
                0.5 * (acc - qsum - ecsq) + bias + xv * w_lin + b_lin)

        pltpu.sync_copy(out_v, out_hbm.at[pl.ds(wid * bpw, bpw)])

    return body(idx3, emb2d, lin1d, x1d, combo)


def kernel(categorical_features, continuous_features, linear_table, bias,
           embedding_table, W_lin, b_lin, W_emb, b_emb):
    B, F = categorical_features.shape
    V, D = embedding_table.shape
    NW = NC * NS
    R = 4
    CH = (B // NW) // R

    idx3 = (categorical_features.astype(jnp.int32)
            .reshape(NW, CH, R, F).transpose(0, 1, 3, 2)
            .reshape(NW, CH, F * R))
    emb2d = lax.optimization_barrier(embedding_table.reshape(-1)).reshape(V, D)
    lin1d = linear_table.reshape(-1)
    x1d = continuous_features.reshape(-1)
    combo = jnp.concatenate([
        W_emb.reshape(-1), b_emb.reshape(-1), W_lin.reshape(-1),
        b_lin.reshape(-1), bias.reshape(-1),
        jnp.zeros((2 * D + 16 - (2 * D + 3),), jnp.float32),
    ])
    return _sc_fm(idx3, emb2d, lin1d, x1d, combo, B, F, D, R)
